# baseline (speedup 1.0000x reference)
import functools

import jax
import jax.numpy as jnp
from jax import lax
from jax.experimental import pallas as pl
from jax.experimental.pallas import tpu as pltpu
from jax.experimental.pallas import tpu_sc as plsc

D = 128
NW = 32
BBLK = 128
RING = 5


def _make_gather(batch: int, fields: int):
    rounds = fields // RING
    mesh = plsc.VectorSubcoreMesh(core_axis_name="c", subcore_axis_name="s")

    @functools.partial(
        pl.kernel,
        mesh=mesh,
        out_type=jax.ShapeDtypeStruct((fields, batch, D), jnp.float32),
        scratch_types=[
            pltpu.VMEM((fields, BBLK), jnp.int32),
            pltpu.VMEM((RING, BBLK, D), jnp.float32),
            [pltpu.SemaphoreType.DMA] * RING,
            [pltpu.SemaphoreType.DMA] * RING,
        ],
        compiler_params=pltpu.CompilerParams(use_tc_tiling_on_sc=True),
    )
    def gather_kernel(idx_hbm, table_hbm, out_hbm, idx_v, rows, gsems, osems):
        wid = lax.axis_index("s") * 2 + lax.axis_index("c")
        b0 = wid * BBLK
        pltpu.sync_copy(idx_hbm.at[:, pl.ds(b0, BBLK)], idx_v)

        def gather(f, k):
            return pltpu.make_async_copy(
                table_hbm.at[idx_v.at[f]], rows.at[k], gsems[k])

        def write(f, k):
            return pltpu.make_async_copy(
                rows.at[k], out_hbm.at[f, pl.ds(b0, BBLK)], osems[k])

        for k in range(RING):
            gather(k, k).start()
        for k in range(RING):
            gather(k, k).wait()
            write(k, k).start()

        def body(h, carry):
            f0 = h * RING
            for k in range(RING):
                write(f0 - RING + k, k).wait()
                gather(f0 + k, k).start()
            for k in range(RING):
                gather(f0 + k, k).wait()
                write(f0 + k, k).start()
            return carry

        lax.fori_loop(1, rounds, body, 0)

        f0 = (rounds - 1) * RING
        for k in range(RING):
            write(f0 + k, k).wait()

    return gather_kernel


def kernel(indices, table):
    batch, fields = indices.shape
    idx_t = indices.T.astype(jnp.int32)
    out_t = _make_gather(batch, fields)(idx_t, table)
    return out_t.transpose(1, 0, 2)


# speedup vs baseline: 1.0253x; 1.0253x over previous
import functools

import jax
import jax.numpy as jnp
from jax import lax
from jax.experimental import pallas as pl
from jax.experimental.pallas import tpu as pltpu
from jax.experimental.pallas import tpu_sc as plsc

D = 128
NW = 32
BBLK = 128
GROUP = 2


def _make_gather(batch: int, fields: int):
    n_groups = fields // GROUP
    half = n_groups // 2
    mesh = plsc.VectorSubcoreMesh(core_axis_name="c", subcore_axis_name="s")

    @functools.partial(
        pl.kernel,
        mesh=mesh,
        out_type=jax.ShapeDtypeStruct((fields, batch, D), jnp.float32),
        scratch_types=[
            pltpu.VMEM((fields, BBLK), jnp.int32),
            pltpu.VMEM((GROUP, BBLK, D), jnp.float32),
            pltpu.VMEM((GROUP, BBLK, D), jnp.float32),
            pltpu.SemaphoreType.DMA,
            pltpu.SemaphoreType.DMA,
        ],
        compiler_params=pltpu.CompilerParams(use_tc_tiling_on_sc=True),
    )
    def gather_kernel(idx_hbm, table_hbm, out_hbm, idx_v, rows_a, rows_b,
                      sem_a, sem_b):
        wid = lax.axis_index("s") * 2 + lax.axis_index("c")
        b0 = wid * BBLK
        pltpu.sync_copy(idx_hbm.at[:, pl.ds(b0, BBLK)], idx_v)

        def gathers(g, buf, sem):
            return [pltpu.make_async_copy(
                        table_hbm.at[idx_v.at[g * GROUP + j]],
                        buf.at[j],
                        sem)
                    for j in range(GROUP)]

        def fire(g, buf, sem):
            for cp in gathers(g, buf, sem):
                cp.start()

        def drain(g, buf, sem):
            for cp in gathers(g, buf, sem):
                cp.wait()

        def write_out(g, buf):
            pltpu.sync_copy(buf, out_hbm.at[pl.ds(g * GROUP, GROUP),
                                            pl.ds(b0, BBLK)])

        fire(0, rows_a, sem_a)

        def body(h, carry):
            g = 2 * h
            fire(g + 1, rows_b, sem_b)
            drain(g, rows_a, sem_a)
            write_out(g, rows_a)
            fire(g + 2, rows_a, sem_a)
            drain(g + 1, rows_b, sem_b)
            write_out(g + 1, rows_b)
            return carry

        lax.fori_loop(0, half - 1, body, 0)

        g = n_groups - 2
        fire(g + 1, rows_b, sem_b)
        drain(g, rows_a, sem_a)
        write_out(g, rows_a)
        drain(g + 1, rows_b, sem_b)
        write_out(g + 1, rows_b)

    return gather_kernel


def kernel(indices, table):
    batch, fields = indices.shape
    idx_t = indices.T.astype(jnp.int32)
    out_t = _make_gather(batch, fields)(idx_t, table)
    return out_t.transpose(1, 0, 2)
